# Initial kernel scaffold; baseline (speedup 1.0000x reference)
import jax
import jax.numpy as jnp
from jax.experimental import pallas as pl


def kernel(xyz, points, points_res):
    raise NotImplementedError("write your pallas kernel here")


# speedup vs baseline: 15.8608x; 15.8608x over previous
import functools

import jax
import jax.numpy as jnp
from jax import lax
from jax.experimental import pallas as pl
from jax.experimental.pallas import tpu as pltpu
from jax.experimental.pallas import tpu_sc as plsc

B = 8
N = 8192
S = N // 4
NC, NS, L = 2, 16, 16
GRP = 4
PPT = N // GRP
NCHUNK = PPT // L
OPT = S // GRP
CH = OPT // 4
D = 128
DOUT = 3 + D
FPAD = PPT + L
SPAD = S + L

_mesh = plsc.VectorSubcoreMesh(
    core_axis_name="c", subcore_axis_name="s", num_cores=NC, num_subcores=NS)

_DNUMS = lax.GatherDimensionNumbers(
    offset_dims=(), collapsed_slice_dims=(0,), start_index_map=(0,))


def _perm(v, ix):
    return lax.gather(v, ix[:, None], dimension_numbers=_DNUMS,
                      slice_sizes=(1,),
                      mode=lax.GatherScatterMode.PROMISE_IN_BOUNDS)


def _srl31(x):
    return lax.shift_right_logical(x, 31)


def _combine(av, ai, bv, bi):
    ab = plsc.bitcast(av, jnp.int32)
    bb = plsc.bitcast(bv, jnp.int32)
    g01 = _srl31(ab - bb)
    x = jnp.bitwise_xor(ab, bb)
    eq01 = 1 - jnp.minimum(x, 1)
    l01 = _srl31(bi - ai)
    take01 = jnp.bitwise_or(g01, jnp.bitwise_and(eq01, l01))
    em = -take01
    nem = jnp.bitwise_not(em)
    nv = jnp.bitwise_or(jnp.bitwise_and(ab, nem), jnp.bitwise_and(bb, em))
    ni = jnp.bitwise_or(jnp.bitwise_and(ai, nem), jnp.bitwise_and(bi, em))
    return plsc.bitcast(nv, jnp.float32), ni, em, nem


def _blendf(a, b, em, nem):
    ab = plsc.bitcast(a, jnp.int32)
    bb = plsc.bitcast(b, jnp.int32)
    return plsc.bitcast(
        jnp.bitwise_or(jnp.bitwise_and(ab, nem), jnp.bitwise_and(bb, em)),
        jnp.float32)


@functools.partial(
    pl.kernel,
    out_type=(
        jax.ShapeDtypeStruct((B * S, 3), jnp.float32),
        jax.ShapeDtypeStruct((B * S, DOUT), jnp.float32),
        jax.ShapeDtypeStruct((B * S, D), jnp.float32),
    ),
    mesh=_mesh,
    compiler_params=pltpu.CompilerParams(
        use_tc_tiling_on_sc=False, needs_layout_passes=False),
    scratch_types=dict(
        xl=pltpu.VMEM((PPT,), jnp.float32),
        yl=pltpu.VMEM((PPT,), jnp.float32),
        zl=pltpu.VMEM((PPT,), jnp.float32),
        dists=pltpu.VMEM((PPT,), jnp.float32),
        flags=pltpu.VMEM((FPAD,), jnp.int32),
        pub=pltpu.VMEM((1, 128), jnp.float32),
        combuf=pltpu.VMEM((GRP, 128), jnp.float32),
        sidx=pltpu.VMEM((SPAD,), jnp.int32),
        sidxg=pltpu.VMEM((SPAD,), jnp.int32),
        flagsall=pltpu.VMEM((GRP * PPT,), jnp.int32),
        xyzall=pltpu.VMEM((GRP * 3 * PPT,), jnp.float32),
        pbuf=pltpu.VMEM((CH, D), jnp.float32),
        rbuf=pltpu.VMEM((CH, DOUT), jnp.float32),
        xbuf=pltpu.VMEM((CH, 3), jnp.float32),
        resbuf=pltpu.VMEM((CH, D), jnp.float32),
        comm=pltpu.VMEM_SHARED((NS, 128), jnp.float32),
        shflags=pltpu.VMEM_SHARED((NS * PPT,), jnp.int32),
        shxyz=pltpu.VMEM_SHARED((NS * 3 * PPT,), jnp.float32),
        sem=pltpu.SemaphoreType.DMA,
        sem2=pltpu.SemaphoreType.DMA,
    ),
)
def _sc_grouper(xtf, pts, res, nxyz_o, npts_o, res_o,
                xl, yl, zl, dists, flags, pub, combuf, sidx, sidxg,
                flagsall, xyzall, pbuf, rbuf, xbuf, resbuf,
                comm, shflags, shxyz, sem, sem2):
    c = lax.axis_index("c")
    s = lax.axis_index("s")
    b = c * 4 + s // GRP
    m = s % GRP
    gbase = (s // GRP) * GRP
    base = m * PPT

    iota = lax.iota(jnp.int32, L)
    zero_v = jnp.zeros((L,), jnp.int32)
    ones_i = jnp.ones((L,), jnp.int32)
    base_v = jnp.full((L,), base, jnp.int32)

    xoff = b * 3 * N + base
    pltpu.sync_copy(xtf.at[pl.ds(xoff, PPT)], xl)
    pltpu.sync_copy(xtf.at[pl.ds(xoff + N, PPT)], yl)
    pltpu.sync_copy(xtf.at[pl.ds(xoff + 2 * N, PPT)], zl)
    pltpu.sync_copy(xl, shxyz.at[pl.ds((s * 3 + 0) * PPT, PPT)])
    pltpu.sync_copy(yl, shxyz.at[pl.ds((s * 3 + 1) * PPT, PPT)])
    pltpu.sync_copy(zl, shxyz.at[pl.ds((s * 3 + 2) * PPT, PPT)])

    def init_body(j, _):
        sl = pl.ds(j * L, L)
        dists[sl] = jnp.full((L,), 1e10, jnp.float32)
        flags[sl] = jnp.zeros((L,), jnp.int32)
        return 0
    lax.fori_loop(0, NCHUNK, init_body, 0)
    flags[pl.ds(PPT, L)] = jnp.zeros((L,), jnp.int32)

    cx0 = _perm(xl[pl.ds(0, L)], zero_v)
    cy0 = _perm(yl[pl.ds(0, L)], zero_v)
    cz0 = _perm(zl[pl.ds(0, L)], zero_v)
    pub[0, pl.ds(32, L)] = cx0
    pub[0, pl.ds(48, L)] = cy0
    pub[0, pl.ds(64, L)] = cz0
    pltpu.sync_copy(pub, comm.at[pl.ds(s, 1)])
    plsc.subcore_barrier()
    pltpu.sync_copy(comm.at[pl.ds(gbase, GRP)], combuf)
    plsc.subcore_barrier()
    wcx = combuf[0, pl.ds(32, L)]
    wcy = combuf[0, pl.ds(48, L)]
    wcz = combuf[0, pl.ds(64, L)]
    wi_v = zero_v

    zero_f = jnp.zeros((L,), jnp.float32)

    def step_body(i, carry):
        wi_v, wcx, wcy, wcz = carry
        loc_v = wi_v - base_v
        neg01 = _srl31(loc_v)
        hi01 = _srl31((PPT - 1) - loc_v)
        own01 = jnp.bitwise_and(1 - neg01, 1 - hi01)
        emo = -own01
        addr = jnp.bitwise_or(
            jnp.bitwise_and(loc_v, emo),
            jnp.bitwise_and(jnp.full((L,), PPT, jnp.int32),
                            jnp.bitwise_not(emo)))
        plsc.store_scatter(flags, [addr], ones_i)

        def chunk(j, acc):
            bmax, bidx, bx, by, bz = acc
            sl = pl.ds(j * L, L)
            xv = xl[sl]
            yv = yl[sl]
            zv = zl[sl]
            dx = xv - wcx
            dy = yv - wcy
            dz = zv - wcz
            d = dx * dx + dy * dy
            d = d + dz * dz
            nd = jnp.minimum(dists[sl], d)
            dists[sl] = nd
            gidx = base + j * L + iota
            newmax = jnp.maximum(bmax, nd)
            diff = (plsc.bitcast(newmax, jnp.int32)
                    - plsc.bitcast(bmax, jnp.int32))
            em = -jnp.minimum(diff, 1)
            nem = jnp.bitwise_not(em)
            bidx = jnp.bitwise_or(jnp.bitwise_and(bidx, nem),
                                  jnp.bitwise_and(gidx, em))
            bx = _blendf(bx, xv, em, nem)
            by = _blendf(by, yv, em, nem)
            bz = _blendf(bz, zv, em, nem)
            return (newmax, bidx, bx, by, bz)

        bmax, bidx, bx, by, bz = lax.fori_loop(
            0, NCHUNK, chunk, (zero_f, zero_v, zero_f, zero_f, zero_f))
        for dsh in (1, 2, 4, 8):
            pix = jnp.bitwise_xor(iota, dsh)
            pv = _perm(bmax, pix)
            pi = _perm(bidx, pix)
            px = _perm(bx, pix)
            py = _perm(by, pix)
            pz = _perm(bz, pix)
            bmax, bidx, em, nem = _combine(bmax, bidx, pv, pi)
            bx = _blendf(bx, px, em, nem)
            by = _blendf(by, py, em, nem)
            bz = _blendf(bz, pz, em, nem)
        pub[0, pl.ds(0, L)] = bmax
        pub[0, pl.ds(16, L)] = plsc.bitcast(bidx, jnp.float32)
        pub[0, pl.ds(32, L)] = bx
        pub[0, pl.ds(48, L)] = by
        pub[0, pl.ds(64, L)] = bz
        pltpu.sync_copy(pub, comm.at[pl.ds(s, 1)])
        plsc.subcore_barrier()
        pltpu.sync_copy(comm.at[pl.ds(gbase, GRP)], combuf)
        plsc.subcore_barrier()
        bv = combuf[0, pl.ds(0, L)]
        bi = plsc.bitcast(combuf[0, pl.ds(16, L)], jnp.int32)
        bcx = combuf[0, pl.ds(32, L)]
        bcy = combuf[0, pl.ds(48, L)]
        bcz = combuf[0, pl.ds(64, L)]
        for t in range(1, GRP):
            tv = combuf[t, pl.ds(0, L)]
            ti = plsc.bitcast(combuf[t, pl.ds(16, L)], jnp.int32)
            bv, bi, em, nem = _combine(bv, bi, tv, ti)
            bcx = _blendf(bcx, combuf[t, pl.ds(32, L)], em, nem)
            bcy = _blendf(bcy, combuf[t, pl.ds(48, L)], em, nem)
            bcz = _blendf(bcz, combuf[t, pl.ds(64, L)], em, nem)
        return (bi, bcx, bcy, bcz)

    lax.fori_loop(0, S, step_body, (wi_v, wcx, wcy, wcz))

    pltpu.sync_copy(flags.at[pl.ds(0, PPT)], shflags.at[pl.ds(s * PPT, PPT)])
    plsc.subcore_barrier()
    pltpu.sync_copy(shflags.at[pl.ds(gbase * PPT, GRP * PPT)], flagsall)
    pltpu.sync_copy(shxyz.at[pl.ds(gbase * 3 * PPT, GRP * 3 * PPT)], xyzall)

    boff = jnp.full((L,), b * N, jnp.int32)
    lane15 = jnp.full((L,), 15, jnp.int32)
    sdump = jnp.full((L,), S, jnp.int32)

    def cbody(jc, pos_v):
        sl = pl.ds(jc * L, L)
        fv = flagsall[sl]
        pc = fv
        for dsh in (1, 2, 4, 8):
            sh = _perm(pc, jnp.maximum(iota - dsh, 0))
            ok01 = 1 - _srl31(iota - dsh)
            pc = pc + jnp.bitwise_and(sh, -ok01)
        positions = pos_v + pc - 1
        emf = -fv
        addr = jnp.bitwise_or(
            jnp.bitwise_and(positions, emf),
            jnp.bitwise_and(sdump, jnp.bitwise_not(emf)))
        gidx = jc * L + iota
        plsc.store_scatter(sidx, [addr], gidx)
        plsc.store_scatter(sidxg, [addr], gidx + boff)
        return pos_v + _perm(pc, lane15)

    lax.fori_loop(0, GRP * NCHUNK, cbody, zero_v)

    for ci in range(4):
        obase = m * OPT + ci * CH
        idxg_ref = sidxg.at[pl.ds(obase, CH)]
        cp1 = pltpu.async_copy(pts.at[idxg_ref], pbuf, sem)
        cp2 = pltpu.async_copy(res.at[idxg_ref], resbuf, sem2)
        for q in range(CH // L):
            iv = sidx[pl.ds(obase + q * L, L)]
            tv = lax.shift_right_logical(iv, 11)
            ov = jnp.bitwise_and(iv, PPT - 1)
            fbase = (lax.shift_left(tv, 12) + lax.shift_left(tv, 11)) + ov
            xv = plsc.load_gather(xyzall, [fbase])
            yv = plsc.load_gather(xyzall, [fbase + PPT])
            zv = plsc.load_gather(xyzall, [fbase + 2 * PPT])
            rv = q * L + iota
            plsc.store_scatter(xbuf, [rv, zero_v], xv)
            plsc.store_scatter(xbuf, [rv, ones_i], yv)
            plsc.store_scatter(xbuf, [rv, ones_i + 1], zv)
            plsc.store_scatter(rbuf, [rv, zero_v], xv)
            plsc.store_scatter(rbuf, [rv, ones_i], yv)
            plsc.store_scatter(rbuf, [rv, ones_i + 1], zv)
        cp1.wait()

        def rowbody(r, _):
            rvv = jnp.full((L,), r, jnp.int32)
            for k in range(D // L):
                cv = k * L + iota
                v = plsc.load_gather(pbuf, [rvv, cv])
                plsc.store_scatter(rbuf, [rvv, cv + 3], v)
            return 0
        lax.fori_loop(0, CH, rowbody, 0)
        cp2.wait()

        ob = b * S + obase
        pltpu.sync_copy(xbuf, nxyz_o.at[pl.ds(ob, CH)])
        pltpu.sync_copy(rbuf, npts_o.at[pl.ds(ob, CH)])
        pltpu.sync_copy(resbuf, res_o.at[pl.ds(ob, CH)])


def kernel(xyz, points, points_res):
    xtf = jnp.transpose(xyz, (0, 2, 1)).reshape(-1)
    pts = points.reshape(B * N, D)
    res = points_res.reshape(B * N, D)
    nxyz, npts, resg = _sc_grouper(xtf, pts, res)
    return (nxyz.reshape(B, S, 3),
            npts.reshape(B, S, 1, DOUT),
            resg.reshape(B, S, D))
